# Initial kernel scaffold; baseline (speedup 1.0000x reference)
import jax
import jax.numpy as jnp
from jax.experimental import pallas as pl


def kernel(x, flat, adjs, batch_size, edge_weight, Wih, Whh, b_lstm, W_self, W_neigh, b_gnn, W_fc, b_fc, W_lo, b_lo):
    raise NotImplementedError("write your pallas kernel here")


# speedup vs baseline: 10.9054x; 10.9054x over previous
import functools

import jax
import jax.numpy as jnp
from jax import lax
from jax.experimental import pallas as pl
from jax.experimental.pallas import tpu as pltpu
from jax.experimental.pallas import tpu_sc as plsc

N = 10000
E = 640000
T = 4
D_IN = 64
H = 32
XD = T * H
F = 64
FLAT_D = 64

NB = 10
BN = N // NB

NTILES = 32
EPT = E // NTILES
K = 80
NW = EPT // K
NPAD = 10240
RPT = NPAD // 16
DPT = NPAD // 16


def _tc1_body(x_ref, flat_ref, wih_ref, whh_ref, b_ref, wn_ref, ws_ref,
              wfcf_ref, wfcl_ref, bfc_ref, wlo_ref, blo_ref,
              u_ref, s_ref, p_ref, ylstm_ref):
    x = x_ref[...]
    h = jnp.zeros((BN, H), jnp.float32)
    c = jnp.zeros((BN, H), jnp.float32)
    hs = []
    for t in range(T):
        z = (jnp.dot(x[:, t * D_IN:(t + 1) * D_IN], wih_ref[...],
                     preferred_element_type=jnp.float32)
             + jnp.dot(h, whh_ref[...], preferred_element_type=jnp.float32)
             + b_ref[...])
        i = jax.nn.sigmoid(z[:, 0:H])
        f = jax.nn.sigmoid(z[:, H:2 * H])
        g = jnp.tanh(z[:, 2 * H:3 * H])
        o = jax.nn.sigmoid(z[:, 3 * H:4 * H])
        c = f * c + i * g
        h = o * jnp.tanh(c)
        hs.append(h)
    xf = jnp.concatenate(hs, axis=1)
    last = hs[-1]
    u_ref[...] = jnp.dot(xf, wn_ref[...], preferred_element_type=jnp.float32)
    s_ref[...] = jnp.dot(xf, ws_ref[...], preferred_element_type=jnp.float32)
    p_ref[...] = (jnp.dot(flat_ref[...], wfcf_ref[...],
                          preferred_element_type=jnp.float32)
                  + jnp.dot(last, wfcl_ref[...],
                            preferred_element_type=jnp.float32)
                  + bfc_ref[...])
    ylstm_ref[...] = jax.nn.sigmoid(
        jnp.dot(last, wlo_ref[...], preferred_element_type=jnp.float32)
        + blo_ref[...])


def _tc1(x2, flat, wih_t, whh_t, b2, w_neigh, w_self, wfc_f, wfc_l, bfc2,
         w_lo, blo2):
    full = lambda shape: pl.BlockSpec(shape, lambda i: tuple(0 for _ in shape))
    return pl.pallas_call(
        _tc1_body,
        grid=(NB,),
        in_specs=[
            pl.BlockSpec((BN, T * D_IN), lambda i: (i, 0)),
            pl.BlockSpec((BN, FLAT_D), lambda i: (i, 0)),
            full((D_IN, 4 * H)),
            full((H, 4 * H)),
            full((1, 4 * H)),
            full((XD, F)),
            full((XD, F)),
            full((FLAT_D, 1)),
            full((H, 1)),
            full((1, 1)),
            full((H, 1)),
            full((1, 1)),
        ],
        out_specs=[
            pl.BlockSpec((BN, F), lambda i: (i, 0)),
            pl.BlockSpec((BN, F), lambda i: (i, 0)),
            pl.BlockSpec((BN, 1), lambda i: (i, 0)),
            pl.BlockSpec((BN, 1), lambda i: (i, 0)),
        ],
        out_shape=[
            jax.ShapeDtypeStruct((N, F), jnp.float32),
            jax.ShapeDtypeStruct((N, F), jnp.float32),
            jax.ShapeDtypeStruct((N, 1), jnp.float32),
            jax.ShapeDtypeStruct((N, 1), jnp.float32),
        ],
    )(x2, flat, wih_t, whh_t, b2, w_neigh, w_self, wfc_f, wfc_l, bfc2,
      w_lo, blo2)


def _sc_body(u_hbm, src_hbm, dst_hbm, w_hbm, zf_hbm, zd_hbm,
             acc_out, deg_out,
             src_t, dst_t, w_t, gbuf,
             acc_sp, deg_sp):
    cid = lax.axis_index("c")
    sid = lax.axis_index("s")
    tid = cid * 16 + sid

    pltpu.sync_copy(zf_hbm.at[pl.ds(sid * RPT, RPT)],
                    acc_sp.at[pl.ds(sid * RPT, RPT)])
    pltpu.sync_copy(zd_hbm.at[pl.ds(sid * DPT, DPT)],
                    deg_sp.at[pl.ds(sid * DPT, DPT)])
    pltpu.sync_copy(src_hbm.at[tid], src_t)
    pltpu.sync_copy(dst_hbm.at[tid], dst_t)
    pltpu.sync_copy(w_hbm.at[tid], w_t)
    plsc.subcore_barrier()

    @pl.loop(0, NW)
    def _window(j):
        pltpu.sync_copy(u_hbm.at[src_t.at[j]], gbuf)
        jk = j * K
        for g in range(K // 16):
            w16 = w_t[pl.ds(jk + g * 16, 16)]
            for l in range(16):
                e = g * 16 + l
                wb = lax.gather(
                    w16, jnp.full((16, 1), l, jnp.int32),
                    lax.GatherDimensionNumbers(
                        offset_dims=(), collapsed_slice_dims=(0,),
                        start_index_map=(0,)),
                    (1,), mode=lax.GatherScatterMode.PROMISE_IN_BOUNDS)
                for q in range(F // 16):
                    sl = pl.ds(q * 16, 16)
                    gbuf[e, sl] = gbuf[e, sl] * wb
        pltpu.sync_copy(gbuf, acc_sp.at[dst_t.at[j]], add=True)
        pltpu.sync_copy(w_t.at[pl.ds(jk, K)], deg_sp.at[dst_t.at[j]],
                        add=True)

    plsc.subcore_barrier()
    pltpu.sync_copy(acc_sp.at[pl.ds(sid * RPT, RPT)],
                    acc_out.at[cid, pl.ds(sid * RPT, RPT)])
    pltpu.sync_copy(deg_sp.at[pl.ds(sid * DPT, DPT)],
                    deg_out.at[cid, pl.ds(sid * DPT, DPT)])


@functools.partial(jax.jit, static_argnums=())
def _sc_seg(u, src_r, dst_r, w_r, zf, zd):
    return pl.kernel(
        _sc_body,
        out_type=[
            jax.ShapeDtypeStruct((2, NPAD, F), jnp.float32),
            jax.ShapeDtypeStruct((2, NPAD), jnp.float32),
        ],
        mesh=plsc.VectorSubcoreMesh(core_axis_name="c", subcore_axis_name="s"),
        compiler_params=pltpu.CompilerParams(use_tc_tiling_on_sc=False),
        scratch_types=[
            pltpu.VMEM((NW, K), jnp.int32),
            pltpu.VMEM((NW, K), jnp.int32),
            pltpu.VMEM((EPT,), jnp.float32),
            pltpu.VMEM((K, F), jnp.float32),
            pltpu.VMEM_SHARED((NPAD, F), jnp.float32),
            pltpu.VMEM_SHARED((NPAD,), jnp.float32),
        ],
    )(u, src_r, dst_r, w_r, zf, zd)


def _tc2_body(s_ref, acc_ref, deg_ref, p_ref, bg_ref, wh_ref, y_ref):
    d = jnp.clip(deg_ref[0] + deg_ref[1], 1e-6, None)
    agg = (acc_ref[0] + acc_ref[1]) / d
    hg = jnp.maximum(s_ref[...] + agg + bg_ref[...], 0.0)
    y_ref[...] = jax.nn.sigmoid(
        jnp.dot(hg, wh_ref[...], preferred_element_type=jnp.float32)
        + p_ref[...])


def _tc2(s, acc, deg, p, bg2, wfc_h):
    return pl.pallas_call(
        _tc2_body,
        grid=(NB,),
        in_specs=[
            pl.BlockSpec((BN, F), lambda i: (i, 0)),
            pl.BlockSpec((2, BN, F), lambda i: (0, i, 0)),
            pl.BlockSpec((2, BN, 1), lambda i: (0, i, 0)),
            pl.BlockSpec((BN, 1), lambda i: (i, 0)),
            pl.BlockSpec((1, F), lambda i: (0, 0)),
            pl.BlockSpec((F, 1), lambda i: (0, 0)),
        ],
        out_specs=pl.BlockSpec((BN, 1), lambda i: (i, 0)),
        out_shape=jax.ShapeDtypeStruct((N, 1), jnp.float32),
    )(s, acc, deg, p, bg2, wfc_h)


def kernel(x, flat, adjs, batch_size, edge_weight, Wih, Whh, b_lstm,
           W_self, W_neigh, b_gnn, W_fc, b_fc, W_lo, b_lo):
    x2 = x.reshape(N, T * D_IN)
    u, s, p, ylstm = _tc1(
        x2, flat, Wih.T, Whh.T, b_lstm.reshape(1, 4 * H), W_neigh, W_self,
        W_fc[F:F + FLAT_D], W_fc[F + FLAT_D:], b_fc.reshape(1, 1),
        W_lo, b_lo.reshape(1, 1))
    src_r = adjs[0].astype(jnp.int32).reshape(NTILES, NW, K)
    dst_r = adjs[1].astype(jnp.int32).reshape(NTILES, NW, K)
    w_r = edge_weight.reshape(NTILES, EPT)
    zf = jnp.zeros((NPAD, F), jnp.float32)
    zd = jnp.zeros((NPAD,), jnp.float32)
    u_p = jnp.pad(u, ((0, NPAD - N), (0, 0)))
    acc, deg = _sc_seg(u_p, src_r, dst_r, w_r, zf, zd)
    y = _tc2(s, acc[:, :N], deg[:, :N].reshape(2, N, 1), p,
             b_gnn.reshape(1, F), W_fc[0:F])
    return (y, ylstm)
